# baseline (speedup 1.0000x reference)
import functools

import jax
import jax.numpy as jnp
from jax import lax
from jax.experimental import pallas as pl
from jax.experimental.pallas import tpu as pltpu
from jax.experimental.pallas import tpu_sc as plsc

DIM = 128
NUM_BUCKETS = 32
N_WORKERS = 32
EDGES_PER_BLOCK = 400


def _fold_kernel(be_ref, w1t_ref, b1r_ref, w2t_ref, b2r_ref, wct_ref,
                 bcr_ref, u_ref):
    nb1 = float(NUM_BUCKETS - 1)
    a = wct_ref[0:DIM, :]
    b = wct_ref[DIM:2 * DIM, :]
    t = jnp.dot(be_ref[:], a, preferred_element_type=jnp.float32)
    m = jnp.dot(w2t_ref[:], b, preferred_element_type=jnp.float32)
    q = jax.lax.broadcasted_iota(jnp.int32, (NUM_BUCKETS, DIM),
                                 0).astype(jnp.float32)
    nodes_lo = q / nb1
    nodes_hi = (q + 1.0) / nb1
    s_lo = nodes_lo * w1t_ref[:] + b1r_ref[:]
    s_hi = nodes_hi * w1t_ref[:] + b1r_ref[:]
    h_lo = s_lo * jax.nn.sigmoid(s_lo)
    h_hi = s_hi * jax.nn.sigmoid(s_hi)
    g_lo = jnp.dot(h_lo, m, preferred_element_type=jnp.float32)
    g_hi = jnp.dot(h_hi, m, preferred_element_type=jnp.float32)
    a1 = (g_hi - g_lo) * nb1
    bias = jnp.dot(b2r_ref[:], b,
                   preferred_element_type=jnp.float32) + bcr_ref[:]
    u_ref[0:NUM_BUCKETS, :] = t + (g_lo + g_hi) * 0.5 + bias
    u_ref[NUM_BUCKETS:2 * NUM_BUCKETS, :] = a1


def _sc_body(pos_hbm, u_hbm, out_hbm, u_v, p_v, out_v, sem):
    nb1 = jnp.float32(NUM_BUCKETS - 1)
    wid = lax.axis_index("s") * 2 + lax.axis_index("c")
    n_per_w = pos_hbm.shape[0] // N_WORKERS
    base = wid * n_per_w
    e = EDGES_PER_BLOCK
    pltpu.sync_copy(u_hbm, u_v)
    lane = lax.iota(jnp.int32, 16)

    def block(g, _):
        off = base + g * e
        pltpu.sync_copy(pos_hbm.at[pl.ds(off, e)], p_v)

        def group(t, _):
            p16 = p_v[pl.ds(t * 16, 16)]
            fi = lax.min(
                lax.max((p16 * nb1).astype(jnp.int32), jnp.int32(0)),
                jnp.int32(NUM_BUCKETS - 1))
            dp = p16 - (fi.astype(jnp.float32) + 0.5) * (1.0 / 31.0)
            rows_hi = fi + NUM_BUCKETS
            e16 = t * 16 + lane
            for c in range(DIM):
                c16 = jnp.full((16,), c, jnp.int32)
                g0 = plsc.load_gather(u_v, [fi, c16])
                g1 = plsc.load_gather(u_v, [rows_hi, c16])
                plsc.store_scatter(out_v, [e16, c16], g0 + dp * g1)
            return 0

        lax.fori_loop(0, e // 16, group, 0)
        pltpu.sync_copy(out_v, out_hbm.at[pl.ds(off, e)])
        return 0

    lax.fori_loop(0, n_per_w // e, block, 0)


@jax.jit
def kernel(positions, bucket_embed, W1, b1, W2, b2, Wc, bc):
    n = positions.shape[0]
    w1t = W1.reshape(1, DIM)
    b1r = b1.reshape(1, DIM)
    w2t = W2.T
    b2r = b2.reshape(1, DIM)
    wct = Wc.T
    bcr = bc.reshape(1, DIM)

    u = pl.pallas_call(
        _fold_kernel,
        out_shape=jax.ShapeDtypeStruct((2 * NUM_BUCKETS, DIM), jnp.float32),
    )(bucket_embed, w1t, b1r, w2t, b2r, wct, bcr)

    mesh = plsc.VectorSubcoreMesh(core_axis_name="c", subcore_axis_name="s")
    sc = functools.partial(
        pl.kernel,
        out_type=jax.ShapeDtypeStruct((n, DIM), jnp.float32),
        mesh=mesh,
        compiler_params=pltpu.CompilerParams(needs_layout_passes=False),
        scratch_types=[
            pltpu.VMEM((2 * NUM_BUCKETS, DIM), jnp.float32),
            pltpu.VMEM((EDGES_PER_BLOCK,), jnp.float32),
            pltpu.VMEM((EDGES_PER_BLOCK, DIM), jnp.float32),
            pltpu.SemaphoreType.DMA,
        ],
    )(_sc_body)
    return sc(positions, u)


# speedup vs baseline: 42.6900x; 42.6900x over previous
import jax
import jax.numpy as jnp
from jax.experimental import pallas as pl
from jax.experimental.pallas import tpu as pltpu

DIM = 128
NUM_BUCKETS = 32
EDGE_BLOCK = 16384
CHUNKS = EDGE_BLOCK // DIM


def _fused_kernel(pos_ref, be_ref, w1t_ref, b1r_ref, w2t_ref, b2r_ref,
                  wct_ref, bcr_ref, out_ref, u_s):
    nb1 = float(NUM_BUCKETS - 1)

    @pl.when(pl.program_id(0) == 0)
    def _fold():
        a = wct_ref[0:DIM, :]
        b = wct_ref[DIM:2 * DIM, :]
        t = jnp.dot(be_ref[:], a, preferred_element_type=jnp.float32)
        m = jnp.dot(w2t_ref[:], b, preferred_element_type=jnp.float32)
        q = jax.lax.broadcasted_iota(jnp.int32, (NUM_BUCKETS, DIM),
                                     0).astype(jnp.float32)
        nodes_lo = q / nb1
        nodes_hi = (q + 1.0) / nb1
        s_lo = nodes_lo * w1t_ref[:] + b1r_ref[:]
        s_hi = nodes_hi * w1t_ref[:] + b1r_ref[:]
        h_lo = s_lo * jax.nn.sigmoid(s_lo)
        h_hi = s_hi * jax.nn.sigmoid(s_hi)
        g_lo = jnp.dot(h_lo, m, preferred_element_type=jnp.float32)
        g_hi = jnp.dot(h_hi, m, preferred_element_type=jnp.float32)
        a1 = (g_hi - g_lo) * nb1
        u0c = t + (g_lo + g_hi) * 0.5
        bias = jnp.dot(b2r_ref[:], b,
                       preferred_element_type=jnp.float32) + bcr_ref[:]
        u_s[0:NUM_BUCKETS, :] = u0c
        u_s[NUM_BUCKETS:2 * NUM_BUCKETS, :] = a1
        u_s[2 * NUM_BUCKETS:2 * NUM_BUCKETS + 1, :] = bias
        u_s[2 * NUM_BUCKETS + 1:DIM, :] = jnp.zeros(
            (DIM - 2 * NUM_BUCKETS - 1, DIM), jnp.float32)

    q = pos_ref[:]
    fq = jnp.clip(jnp.floor(q * nb1), 0.0, nb1)
    dq = q - (fq + 0.5) * (1.0 / nb1)
    fi = jax.lax.broadcasted_iota(jnp.int32, (DIM, 1), 0)
    lm = jnp.where(fi < 2 * NUM_BUCKETS, fi % NUM_BUCKETS,
                   -1).astype(jnp.float32)
    is_lo = fi < NUM_BUCKETS
    miss = (fi == 2 * NUM_BUCKETS).astype(jnp.float32)
    u = u_s[:]

    for r in range(CHUNKS):
        frow = fq[r:r + 1, :]
        drow = dq[r:r + 1, :]
        blend = jnp.where(is_lo, jnp.float32(1.0), drow)
        phi_t = jnp.where(frow == lm, blend, miss)
        out_ref[r * DIM:(r + 1) * DIM, :] = jax.lax.dot_general(
            phi_t, u, (((0,), (0,)), ((), ())),
            preferred_element_type=jnp.float32)


@jax.jit
def kernel(positions, bucket_embed, W1, b1, W2, b2, Wc, bc):
    n = positions.shape[0]
    pos_packed = positions.reshape(n // DIM, DIM)
    w1t = W1.reshape(1, DIM)
    b1r = b1.reshape(1, DIM)
    w2t = W2.T
    b2r = b2.reshape(1, DIM)
    wct = Wc.T
    bcr = bc.reshape(1, DIM)

    grid = pl.cdiv(n, EDGE_BLOCK)
    out = pl.pallas_call(
        _fused_kernel,
        grid=(grid,),
        in_specs=[
            pl.BlockSpec((CHUNKS, DIM), lambda g: (g, 0)),
            pl.BlockSpec((NUM_BUCKETS, DIM), lambda g: (0, 0)),
            pl.BlockSpec((1, DIM), lambda g: (0, 0)),
            pl.BlockSpec((1, DIM), lambda g: (0, 0)),
            pl.BlockSpec((DIM, DIM), lambda g: (0, 0)),
            pl.BlockSpec((1, DIM), lambda g: (0, 0)),
            pl.BlockSpec((2 * DIM, DIM), lambda g: (0, 0)),
            pl.BlockSpec((1, DIM), lambda g: (0, 0)),
        ],
        out_specs=pl.BlockSpec((EDGE_BLOCK, DIM), lambda g: (g, 0)),
        out_shape=jax.ShapeDtypeStruct((n, DIM), jnp.float32),
        scratch_shapes=[
            pltpu.VMEM((DIM, DIM), jnp.float32),
        ],
        compiler_params=pltpu.CompilerParams(
            dimension_semantics=("arbitrary",)),
    )(pos_packed, bucket_embed, w1t, b1r, w2t, b2r, wct, bcr)
    return out
